# baseline (speedup 1.0000x reference)
import functools

import jax
import jax.numpy as jnp
from jax import lax
from jax.experimental import pallas as pl
from jax.experimental.pallas import tpu as pltpu
from jax.experimental.pallas import tpu_sc as plsc

CH = 16


def _sc_add_kernel(B, S, H, NC, NS):
    NW = NC * NS
    rows_per_w = S // NW
    n_chunks = rows_per_w // CH
    mesh = plsc.VectorSubcoreMesh(core_axis_name="c", subcore_axis_name="s")

    @functools.partial(
        pl.kernel,
        mesh=mesh,
        out_type=jax.ShapeDtypeStruct((B, S, H), jnp.float32),
        scratch_types=[
            pltpu.VMEM((CH, H), jnp.float32),
            pltpu.VMEM((CH, H), jnp.float32),
            pltpu.VMEM((CH, H), jnp.float32),
            pltpu.VMEM((CH, H), jnp.float32),
            pltpu.VMEM((CH, H), jnp.float32),
            pltpu.VMEM((CH, H), jnp.float32),
            pltpu.SemaphoreType.DMA,
            pltpu.SemaphoreType.DMA,
            pltpu.SemaphoreType.DMA,
            pltpu.SemaphoreType.DMA,
            pltpu.SemaphoreType.DMA,
            pltpu.SemaphoreType.DMA,
            pltpu.SemaphoreType.DMA,
            pltpu.SemaphoreType.DMA,
            pltpu.SemaphoreType.DMA,
            pltpu.SemaphoreType.DMA,
        ],
    )
    def k(x_hbm, pe_hbm, out_hbm, pb0, pb1, xb0, xb1, xb2, xb3,
          sp0, sp1, si0, si1, si2, si3, so0, so1, so2, so3):
        wid = lax.axis_index("s") * NC + lax.axis_index("c")
        seq0 = wid * rows_per_w
        pbs, sps = (pb0, pb1), (sp0, sp1)
        xbs, sis, sos = (xb0, xb1, xb2, xb3), (si0, si1, si2, si3), (so0, so1, so2, so3)

        def x_src(b, base):
            return x_hbm.at[b, pl.ds(base, CH)]

        def pe_src(base):
            return pe_hbm.at[pl.ds(base, CH)]

        pltpu.async_copy(pe_src(seq0), pb0, sp0)
        pltpu.async_copy(pe_src(seq0 + CH), pb1, sp1)
        pltpu.async_copy(x_src(0, seq0), xb0, si0)
        pltpu.async_copy(x_src(1, seq0), xb1, si1)

        def half_body(cc, carry):
            for i in range(2):
                c = 2 * cc + i
                base = seq0 + c * CH
                pe_v = pbs[i]
                pltpu.make_async_copy(pe_src(base), pe_v, sps[i]).wait()
                for b in range(B):
                    xb, si = xbs[b], sis[b]
                    q = (b + 2) % B
                    pltpu.make_async_copy(x_src(b, base), xb, si).wait()

                    def row_body(r, carry2, xb=xb, pe_v=pe_v):
                        for j in range(H // 16):
                            sl = pl.ds(j * 16, 16)
                            plsc.addupdate(xb.at[r, sl], pe_v[r, sl])
                        return carry2

                    lax.fori_loop(0, CH, row_body, 0)

                    pltpu.async_copy(xb, out_hbm.at[b, pl.ds(base, CH)], sos[b])

                    if b >= 2:
                        pltpu.make_async_copy(
                            xbs[q], out_hbm.at[q, pl.ds(base, CH)], sos[q]
                        ).wait()
                        nbase = jnp.minimum(base + CH, seq0 + (n_chunks - 1) * CH)
                        pltpu.async_copy(x_src(q, nbase), xbs[q], sis[q])
                    else:

                        @pl.when(c > 0)
                        def _(q=q, base=base):
                            pltpu.make_async_copy(
                                xbs[q], out_hbm.at[q, pl.ds(base - CH, CH)], sos[q]
                            ).wait()
                            nbase = jnp.minimum(base, seq0 + (n_chunks - 1) * CH)
                            pltpu.async_copy(x_src(q, nbase), xbs[q], sis[q])

                        @pl.when(c == 0)
                        def _(q=q, base=base):
                            pltpu.async_copy(x_src(q, base), xbs[q], sis[q])

                nbase = seq0 + jnp.minimum(c + 2, n_chunks - 1) * CH
                pltpu.async_copy(pe_src(nbase), pe_v, sps[i])
            return carry

        lax.fori_loop(0, n_chunks // 2, half_body, 0)

        last = seq0 + (n_chunks - 1) * CH
        pltpu.make_async_copy(xbs[2], out_hbm.at[2, pl.ds(last, CH)], sos[2]).wait()
        pltpu.make_async_copy(xbs[3], out_hbm.at[3, pl.ds(last, CH)], sos[3]).wait()
        pltpu.make_async_copy(x_src(0, last), xbs[0], sis[0]).wait()
        pltpu.make_async_copy(x_src(1, last), xbs[1], sis[1]).wait()
        pltpu.make_async_copy(pe_src(last), pbs[0], sps[0]).wait()
        pltpu.make_async_copy(pe_src(last), pbs[1], sps[1]).wait()

    return k


def kernel(x, pe):
    B, S, H = x.shape
    info = plsc.get_sparse_core_info()
    k = _sc_add_kernel(B, S, H, info.num_cores, info.num_subcores)
    return k(x, pe)


# speedup vs baseline: 1.9481x; 1.9481x over previous
import functools

import jax
import jax.numpy as jnp
from jax import lax
from jax.experimental import pallas as pl
from jax.experimental.pallas import tpu as pltpu
from jax.experimental.pallas import tpu_sc as plsc

CH = 16


def _sc_add_kernel(B, S, H, NC, NS):
    NW = NC * NS
    rows_per_w = S // NW
    n_chunks = rows_per_w // CH
    mesh = plsc.VectorSubcoreMesh(core_axis_name="c", subcore_axis_name="s")

    @functools.partial(
        pl.kernel,
        mesh=mesh,
        out_type=jax.ShapeDtypeStruct((B, S, H), jnp.float32),
        scratch_types=[
            pltpu.VMEM((CH, H), jnp.float32),
            pltpu.VMEM((CH, H), jnp.float32),
            pltpu.VMEM((CH, H), jnp.float32),
            pltpu.VMEM((CH, H), jnp.float32),
            pltpu.VMEM((CH, H), jnp.float32),
            pltpu.VMEM((CH, H), jnp.float32),
            pltpu.SemaphoreType.DMA,
            pltpu.SemaphoreType.DMA,
            pltpu.SemaphoreType.DMA,
            pltpu.SemaphoreType.DMA,
            pltpu.SemaphoreType.DMA,
            pltpu.SemaphoreType.DMA,
            pltpu.SemaphoreType.DMA,
            pltpu.SemaphoreType.DMA,
            pltpu.SemaphoreType.DMA,
            pltpu.SemaphoreType.DMA,
        ],
    )
    def k(x_hbm, pe_hbm, out_hbm, pb0, pb1, xb0, xb1, xb2, xb3,
          sp0, sp1, si0, si1, si2, si3, so0, so1, so2, so3):
        wid = lax.axis_index("s") * NC + lax.axis_index("c")
        seq0 = wid * rows_per_w
        pbs, sps = (pb0, pb1), (sp0, sp1)
        xbs, sis, sos = (xb0, xb1, xb2, xb3), (si0, si1, si2, si3), (so0, so1, so2, so3)

        def x_src(b, base):
            return x_hbm.at[b, pl.ds(base, CH)]

        def pe_src(base):
            return pe_hbm.at[pl.ds(base, CH)]

        pltpu.async_copy(pe_src(seq0), pb0, sp0)
        pltpu.async_copy(pe_src(seq0 + CH), pb1, sp1)
        pltpu.async_copy(x_src(0, seq0), xb0, si0)
        pltpu.async_copy(x_src(1, seq0), xb1, si1)

        def half_body(cc, carry):
            for i in range(2):
                c = 2 * cc + i
                base = seq0 + c * CH
                pe_v = pbs[i]
                pltpu.make_async_copy(pe_src(base), pe_v, sps[i]).wait()
                for b in range(B):
                    xb, si = xbs[b], sis[b]
                    q = (b + 2) % B
                    pltpu.make_async_copy(x_src(b, base), xb, si).wait()

                    def row_body(r, carry2, xb=xb, pe_v=pe_v):
                        @plsc.parallel_loop(0, H // 16, unroll=8)
                        def jloop(j):
                            sl = pl.ds(j * 16, 16)
                            plsc.addupdate(xb.at[r, sl], pe_v[r, sl])

                        return carry2

                    lax.fori_loop(0, CH, row_body, 0)

                    pltpu.async_copy(xb, out_hbm.at[b, pl.ds(base, CH)], sos[b])

                    if b >= 2:
                        pltpu.make_async_copy(
                            xbs[q], out_hbm.at[q, pl.ds(base, CH)], sos[q]
                        ).wait()
                        nbase = jnp.minimum(base + CH, seq0 + (n_chunks - 1) * CH)
                        pltpu.async_copy(x_src(q, nbase), xbs[q], sis[q])
                    else:

                        @pl.when(c > 0)
                        def _(q=q, base=base):
                            pltpu.make_async_copy(
                                xbs[q], out_hbm.at[q, pl.ds(base - CH, CH)], sos[q]
                            ).wait()
                            nbase = jnp.minimum(base, seq0 + (n_chunks - 1) * CH)
                            pltpu.async_copy(x_src(q, nbase), xbs[q], sis[q])

                        @pl.when(c == 0)
                        def _(q=q, base=base):
                            pltpu.async_copy(x_src(q, base), xbs[q], sis[q])

                nbase = seq0 + jnp.minimum(c + 2, n_chunks - 1) * CH
                pltpu.async_copy(pe_src(nbase), pe_v, sps[i])
            return carry

        lax.fori_loop(0, n_chunks // 2, half_body, 0)

        last = seq0 + (n_chunks - 1) * CH
        pltpu.make_async_copy(xbs[2], out_hbm.at[2, pl.ds(last, CH)], sos[2]).wait()
        pltpu.make_async_copy(xbs[3], out_hbm.at[3, pl.ds(last, CH)], sos[3]).wait()
        pltpu.make_async_copy(x_src(0, last), xbs[0], sis[0]).wait()
        pltpu.make_async_copy(x_src(1, last), xbs[1], sis[1]).wait()
        pltpu.make_async_copy(pe_src(last), pbs[0], sps[0]).wait()
        pltpu.make_async_copy(pe_src(last), pbs[1], sps[1]).wait()

    return k


def kernel(x, pe):
    B, S, H = x.shape
    info = plsc.get_sparse_core_info()
    k = _sc_add_kernel(B, S, H, info.num_cores, info.num_subcores)
    return k(x, pe)


# speedup vs baseline: 2.0813x; 1.0684x over previous
import functools

import jax
import jax.numpy as jnp
from jax import lax
from jax.experimental import pallas as pl
from jax.experimental.pallas import tpu as pltpu
from jax.experimental.pallas import tpu_sc as plsc

CH = 16


def _sc_add_kernel(B, S, H, NC, NS):
    NW = NC * NS
    rows_per_w = S // NW
    n_chunks = rows_per_w // CH
    mesh = plsc.VectorSubcoreMesh(core_axis_name="c", subcore_axis_name="s")

    @functools.partial(
        pl.kernel,
        mesh=mesh,
        out_type=jax.ShapeDtypeStruct((B, S, H), jnp.float32),
        scratch_types=[
            pltpu.VMEM((CH, H), jnp.float32),
            pltpu.VMEM((CH, H), jnp.float32),
            pltpu.VMEM((CH, H), jnp.float32),
            pltpu.VMEM((CH, H), jnp.float32),
            pltpu.VMEM((CH, H), jnp.float32),
            pltpu.VMEM((CH, H), jnp.float32),
            pltpu.SemaphoreType.DMA,
            pltpu.SemaphoreType.DMA,
            pltpu.SemaphoreType.DMA,
            pltpu.SemaphoreType.DMA,
            pltpu.SemaphoreType.DMA,
            pltpu.SemaphoreType.DMA,
            pltpu.SemaphoreType.DMA,
            pltpu.SemaphoreType.DMA,
            pltpu.SemaphoreType.DMA,
            pltpu.SemaphoreType.DMA,
        ],
    )
    def k(x_hbm, pe_hbm, out_hbm, pb0, pb1, xb0, xb1, xb2, xb3,
          sp0, sp1, si0, si1, si2, si3, so0, so1, so2, so3):
        wid = lax.axis_index("s") * NC + lax.axis_index("c")
        seq0 = wid * rows_per_w
        pbs, sps = (pb0, pb1), (sp0, sp1)
        xbs, sis, sos = (xb0, xb1, xb2, xb3), (si0, si1, si2, si3), (so0, so1, so2, so3)

        def x_src(b, base):
            return x_hbm.at[b, pl.ds(base, CH)]

        def pe_src(base):
            return pe_hbm.at[pl.ds(base, CH)]

        pltpu.async_copy(pe_src(seq0), pb0, sp0)
        pltpu.async_copy(pe_src(seq0 + CH), pb1, sp1)
        pltpu.async_copy(x_src(0, seq0), xb0, si0)
        pltpu.async_copy(x_src(1, seq0), xb1, si1)

        def half_body(cc, carry):
            for i in range(2):
                c = 2 * cc + i
                base = seq0 + c * CH
                pe_v = pbs[i]
                pltpu.make_async_copy(pe_src(base), pe_v, sps[i]).wait()
                for b in range(B):
                    xb, si = xbs[b], sis[b]
                    q = (b + 2) % B
                    pltpu.make_async_copy(x_src(b, base), xb, si).wait()

                    if b >= 2:
                        pltpu.make_async_copy(
                            xbs[q], out_hbm.at[q, pl.ds(base, CH)], sos[q]
                        ).wait()
                        nbase = jnp.minimum(base + CH, seq0 + (n_chunks - 1) * CH)
                        pltpu.async_copy(x_src(q, nbase), xbs[q], sis[q])
                    else:

                        @pl.when(c > 0)
                        def _(q=q, base=base):
                            pltpu.make_async_copy(
                                xbs[q], out_hbm.at[q, pl.ds(base - CH, CH)], sos[q]
                            ).wait()
                            pltpu.async_copy(x_src(q, base), xbs[q], sis[q])

                        @pl.when(c == 0)
                        def _(q=q, base=base):
                            pltpu.async_copy(x_src(q, base), xbs[q], sis[q])

                    def row_body(r, carry2, xb=xb, pe_v=pe_v):
                        @plsc.parallel_loop(0, H // 16, unroll=8)
                        def jloop(j):
                            sl = pl.ds(j * 16, 16)
                            plsc.addupdate(xb.at[r, sl], pe_v[r, sl])

                        return carry2

                    lax.fori_loop(0, CH, row_body, 0)

                    pltpu.async_copy(xb, out_hbm.at[b, pl.ds(base, CH)], sos[b])

                nbase = seq0 + jnp.minimum(c + 2, n_chunks - 1) * CH
                pltpu.async_copy(pe_src(nbase), pe_v, sps[i])
            return carry

        lax.fori_loop(0, n_chunks // 2, half_body, 0)

        last = seq0 + (n_chunks - 1) * CH
        pltpu.make_async_copy(xbs[2], out_hbm.at[2, pl.ds(last, CH)], sos[2]).wait()
        pltpu.make_async_copy(xbs[3], out_hbm.at[3, pl.ds(last, CH)], sos[3]).wait()
        pltpu.make_async_copy(x_src(0, last), xbs[0], sis[0]).wait()
        pltpu.make_async_copy(x_src(1, last), xbs[1], sis[1]).wait()
        pltpu.make_async_copy(pe_src(last), pbs[0], sps[0]).wait()
        pltpu.make_async_copy(pe_src(last), pbs[1], sps[1]).wait()

    return k


def kernel(x, pe):
    B, S, H = x.shape
    info = plsc.get_sparse_core_info()
    k = _sc_add_kernel(B, S, H, info.num_cores, info.num_subcores)
    return k(x, pe)


# speedup vs baseline: 2.1372x; 1.0269x over previous
import functools

import jax
import jax.numpy as jnp
from jax import lax
from jax.experimental import pallas as pl
from jax.experimental.pallas import tpu as pltpu
from jax.experimental.pallas import tpu_sc as plsc

CH = 8


def _sc_add_kernel(B, S, H, NC, NS):
    NW = NC * NS
    rows_per_w = S // NW
    n_chunks = rows_per_w // CH
    mesh = plsc.VectorSubcoreMesh(core_axis_name="c", subcore_axis_name="s")

    @functools.partial(
        pl.kernel,
        mesh=mesh,
        out_type=jax.ShapeDtypeStruct((B, S, H), jnp.float32),
        scratch_types=(
            [pltpu.VMEM((CH, H), jnp.float32) for _ in range(2)]
            + [pltpu.VMEM((CH, H), jnp.float32) for _ in range(8)]
            + [pltpu.SemaphoreType.DMA for _ in range(2)]
            + [pltpu.SemaphoreType.DMA for _ in range(8)]
            + [pltpu.SemaphoreType.DMA for _ in range(8)]
        ),
    )
    def k(x_hbm, pe_hbm, out_hbm, *scr):
        pbs = scr[0:2]
        xbs = scr[2:10]
        sps = scr[10:12]
        sis = scr[12:20]
        sos = scr[20:28]
        wid = lax.axis_index("s") * NC + lax.axis_index("c")
        seq0 = wid * rows_per_w

        def x_src(b, base):
            return x_hbm.at[b, pl.ds(base, CH)]

        def pe_src(base):
            return pe_hbm.at[pl.ds(base, CH)]

        pltpu.async_copy(pe_src(seq0), pbs[0], sps[0])
        pltpu.async_copy(pe_src(seq0 + CH), pbs[1], sps[1])
        for b in range(B):
            pltpu.async_copy(x_src(b, seq0), xbs[b], sis[b])

        def half_body(cc, carry):
            for i in range(2):
                c = 2 * cc + i
                base = seq0 + c * CH
                pe_v = pbs[i]
                pltpu.make_async_copy(pe_src(base), pe_v, sps[i]).wait()
                for b in range(B):
                    p = i * 4 + b
                    pn = (1 - i) * 4 + b
                    xb = xbs[p]
                    pltpu.make_async_copy(x_src(b, base), xb, sis[p]).wait()

                    @pl.when(c > 0)
                    def _(b=b, pn=pn, base=base):
                        pltpu.make_async_copy(
                            xbs[pn], out_hbm.at[b, pl.ds(base - CH, CH)], sos[pn]
                        ).wait()
                        nbase = jnp.minimum(base + CH, seq0 + (n_chunks - 1) * CH)
                        pltpu.async_copy(x_src(b, nbase), xbs[pn], sis[pn])

                    @pl.when(c == 0)
                    def _(b=b, pn=pn, base=base):
                        pltpu.async_copy(x_src(b, base + CH), xbs[pn], sis[pn])

                    def row_body(r, carry2, xb=xb, pe_v=pe_v):
                        @plsc.parallel_loop(0, H // 16, unroll=8)
                        def jloop(j):
                            sl = pl.ds(j * 16, 16)
                            plsc.addupdate(xb.at[r, sl], pe_v[r, sl])

                        return carry2

                    lax.fori_loop(0, CH, row_body, 0)

                    pltpu.async_copy(xb, out_hbm.at[b, pl.ds(base, CH)], sos[p])

                nbase = seq0 + jnp.minimum(c + 2, n_chunks - 1) * CH
                pltpu.async_copy(pe_src(nbase), pe_v, sps[i])
            return carry

        lax.fori_loop(0, n_chunks // 2, half_body, 0)

        last = seq0 + (n_chunks - 1) * CH
        for b in range(B):
            pltpu.make_async_copy(
                xbs[4 + b], out_hbm.at[b, pl.ds(last, CH)], sos[4 + b]
            ).wait()
            pltpu.make_async_copy(x_src(b, last), xbs[b], sis[b]).wait()
        pltpu.make_async_copy(pe_src(last), pbs[0], sps[0]).wait()
        pltpu.make_async_copy(pe_src(last), pbs[1], sps[1]).wait()

    return k


def kernel(x, pe):
    B, S, H = x.shape
    info = plsc.get_sparse_core_info()
    k = _sc_add_kernel(B, S, H, info.num_cores, info.num_subcores)
    return k(x, pe)
